# baseline (speedup 1.0000x reference)
import jax
import jax.numpy as jnp
from jax import lax
from jax.experimental import pallas as pl
from jax.experimental.pallas import tpu as pltpu
from jax.experimental.pallas import tpu_sc as plsc

NC = 2
NS = 16
NW = NC * NS
L = 16

B_TOK = 4096
SEQ = 50
HID = 64
HIDP = 128
NVEC = HID // L
CH = 128
NCH = SEQ
NBUF = 2
NITER = NCH // NBUF

MAGIC = 0x5F3759DF


def _splat_i32(v):
    return jnp.full((L,), v, dtype=jnp.int32)


def _two_groups(buf, tbuf, sq, g2):
    iota = lax.iota(jnp.int32, L)
    cb = [p * L + iota for p in range(NVEC)]
    chi = [c >> 3 for c in cb]
    clo = [c & 7 for c in cb]
    for h in range(2):
        base = (g2 * 2 + h) * L
        for r in range(L):
            v = [buf[base + r, pl.ds(p * L, L)] for p in range(NVEC)]
            s16 = (v[0] * v[0] + v[1] * v[1]) + (v[2] * v[2] + v[3] * v[3])
            sq[pl.ds(h * 256 + r * L, L)] = s16
        fbase = (iota << 4) + (h * 256)
        f = [fbase + kk for kk in range(4)]
        four = _splat_i32(4)
        acc = [None] * 4
        for step in range(4):
            for kk in range(4):
                x = plsc.load_gather(sq, [f[kk]])
                acc[kk] = x if step == 0 else acc[kk] + x
                if step < 3:
                    f[kk] = f[kk] + four
        s = (acc[0] + acc[1]) + (acc[2] + acc[3])
        iv = plsc.bitcast(s, jnp.int32)
        y = plsc.bitcast(jnp.full((L,), MAGIC, jnp.int32) - (iv >> 1), jnp.float32)
        half = s * 0.5
        for _ in range(3):
            y = y * (1.5 - half * y * y)
        inv = 1.0 / (s * y + 1e-8)
        for r in range(L):
            ivb = lax.broadcast_in_dim(inv[r], (L,), ())
            rowv = lax.broadcast_in_dim(base + r, (L,), ())
            for p in range(NVEC):
                x = buf[base + r, pl.ds(p * L, L)]
                plsc.store_scatter(tbuf, [chi[p], clo[p], rowv], x * ivb)


def _body(idx_hbm, table_hbm, out_hbm, idx_v, b0, b1,
          t0, t1, sq, g0, g1, w0, w1):
    bufs = [b0, b1]
    tbufs = [t0, t1]
    gsems = [g0, g1]
    wsems = [w0, w1]
    wid = lax.axis_index("s") * NC + lax.axis_index("c")
    pltpu.sync_copy(idx_hbm.at[wid], idx_v)

    for k in range(NBUF - 1):
        pltpu.async_copy(table_hbm.at[idx_v.at[k]], bufs[k], gsems[k])

    def iter_body(t, carry):
        for k in range(NBUF):
            c = t * NBUF + k
            pltpu.make_async_copy(
                table_hbm.at[idx_v.at[c]], bufs[k], gsems[k]
            ).wait()

            n = c + NBUF - 1
            ps = (k + NBUF - 1) % NBUF

            @pl.when(n < NCH)
            def _(k=k, c=c, n=n, ps=ps):
                pltpu.async_copy(table_hbm.at[idx_v.at[n]], bufs[ps], gsems[ps])

            def pair(g2i, cc, _buf=bufs[k], _tbuf=tbufs[k]):
                _two_groups(_buf, _tbuf, sq, g2i)
                return cc

            @pl.when(c >= NBUF)
            def _(k=k, c=c):
                pltpu.make_async_copy(
                    tbufs[k].at[:, :, pl.ds(0, CH)],
                    out_hbm.at[c - NBUF, :, wid],
                    wsems[k],
                ).wait()

            lax.fori_loop(0, CH // (2 * L), pair, 0)
            pltpu.async_copy(
                tbufs[k].at[:, :, pl.ds(0, CH)],
                out_hbm.at[c, :, wid],
                wsems[k],
            )

        return carry

    lax.fori_loop(0, NITER, iter_body, 0)
    for k in range(NBUF):
        c = NCH - NBUF + k
        pltpu.make_async_copy(
            tbufs[k].at[:, :, pl.ds(0, CH)],
            out_hbm.at[c, :, wid],
            wsems[k],
        ).wait()


@jax.jit
def _emb_call(idx, table):
    mesh = plsc.VectorSubcoreMesh(core_axis_name="c", subcore_axis_name="s")
    f = pl.kernel(
        _body,
        out_type=jax.ShapeDtypeStruct((SEQ, 8, NW, 8, CH), jnp.float32),
        mesh=mesh,
        compiler_params=pltpu.CompilerParams(
            needs_layout_passes=False, use_tc_tiling_on_sc=True
        ),
        scratch_types=(
            [pltpu.VMEM((NCH, CH), jnp.int32)]
            + [pltpu.VMEM((CH, HIDP), jnp.float32) for _ in range(NBUF)]
            + [pltpu.VMEM((8, 8, CH + 1), jnp.float32) for _ in range(NBUF)]
            + [pltpu.VMEM((2 * 256,), jnp.float32)]
            + [pltpu.SemaphoreType.DMA for _ in range(2 * NBUF)]
        ),
    )
    return f(idx, table)


def kernel(input_ids, weight):
    idx = input_ids.astype(jnp.int32).T.reshape(SEQ, NW, CH).transpose(1, 0, 2)
    wp = jnp.pad(weight, ((0, 0), (0, HIDP - HID)))
    out5 = _emb_call(idx, wp)
    return out5.transpose(2, 4, 0, 1, 3).reshape(B_TOK, SEQ, HID)


# speedup vs baseline: 1.1831x; 1.1831x over previous
import jax
import jax.numpy as jnp
from jax import lax
from jax.experimental import pallas as pl
from jax.experimental.pallas import tpu as pltpu
from jax.experimental.pallas import tpu_sc as plsc

NC = 2
NS = 16
NW = NC * NS
L = 16

B_TOK = 4096
SEQ = 50
HID = 64
HIDP = 128
NVEC = HID // L
CH = 128
NCH = SEQ
NBUF = 2
NITER = NCH // NBUF

MAGIC = 0x5F3759DF


def _splat_i32(v):
    return jnp.full((L,), v, dtype=jnp.int32)


def _two_groups(buf, tbuf, sq, g2):
    iota = lax.iota(jnp.int32, L)
    cb = [p * L + iota for p in range(NVEC)]
    chi = [c >> 3 for c in cb]
    clo = [c & 7 for c in cb]
    for h in range(2):
        base = (g2 * 2 + h) * L
        for r in range(L):
            v = [buf[base + r, pl.ds(p * L, L)] for p in range(NVEC)]
            s16 = (v[0] * v[0] + v[1] * v[1]) + (v[2] * v[2] + v[3] * v[3])
            sq[pl.ds(h * 256 + r * L, L)] = s16
        fbase = (iota << 4) + (h * 256)
        f = [fbase + kk for kk in range(4)]
        four = _splat_i32(4)
        acc = [None] * 4
        for step in range(4):
            for kk in range(4):
                x = plsc.load_gather(sq, [f[kk]])
                acc[kk] = x if step == 0 else acc[kk] + x
                if step < 3:
                    f[kk] = f[kk] + four
        s = (acc[0] + acc[1]) + (acc[2] + acc[3])
        iv = plsc.bitcast(s, jnp.int32)
        y = plsc.bitcast(jnp.full((L,), MAGIC, jnp.int32) - (iv >> 1), jnp.float32)
        half = s * 0.5
        for _ in range(3):
            y = y * (1.5 - half * y * y)
        inv = 1.0 / (s * y + 1e-8)
        for r in range(L):
            ivb = lax.broadcast_in_dim(inv[r], (L,), ())
            rowv = lax.broadcast_in_dim(base + r, (L,), ())
            for p in range(NVEC):
                x = buf[base + r, pl.ds(p * L, L)]
                plsc.store_scatter(tbuf, [chi[p], clo[p], rowv], x * ivb)


def _body(idx_hbm, table_hbm, out_hbm, idx_v, b0, b1,
          t0, t1, sq, g0, g1, w0, w1):
    bufs = [b0, b1]
    tbufs = [t0, t1]
    gsems = [g0, g1]
    wsems = [w0, w1]
    wid = lax.axis_index("s") * NC + lax.axis_index("c")
    pltpu.sync_copy(idx_hbm.at[wid], idx_v)

    for k in range(NBUF - 1):
        pltpu.async_copy(table_hbm.at[idx_v.at[k]], bufs[k], gsems[k])

    def iter_body(t, carry):
        for k in range(NBUF):
            c = t * NBUF + k
            pltpu.make_async_copy(
                table_hbm.at[idx_v.at[c]], bufs[k], gsems[k]
            ).wait()

            n = c + NBUF - 1
            ps = (k + NBUF - 1) % NBUF

            @pl.when(n < NCH)
            def _(k=k, c=c, n=n, ps=ps):
                pltpu.async_copy(table_hbm.at[idx_v.at[n]], bufs[ps], gsems[ps])

            def pair(g2i, cc, _buf=bufs[k], _tbuf=tbufs[k]):
                _two_groups(_buf, _tbuf, sq, g2i)
                return cc

            @pl.when(c >= NBUF)
            def _(k=k, c=c):
                pltpu.make_async_copy(
                    tbufs[k].at[:, :, pl.ds(0, CH)],
                    out_hbm.at[c - NBUF, :, wid],
                    wsems[k],
                ).wait()

            lax.fori_loop(0, CH // (2 * L), pair, 0)
            pltpu.async_copy(
                tbufs[k].at[:, :, pl.ds(0, CH)],
                out_hbm.at[c, :, wid],
                wsems[k],
            )

        return carry

    lax.fori_loop(0, NITER, iter_body, 0)
    for k in range(NBUF):
        c = NCH - NBUF + k
        pltpu.make_async_copy(
            tbufs[k].at[:, :, pl.ds(0, CH)],
            out_hbm.at[c, :, wid],
            wsems[k],
        ).wait()


@jax.jit
def _emb_call(idx, table):
    mesh = plsc.VectorSubcoreMesh(core_axis_name="c", subcore_axis_name="s")
    f = pl.kernel(
        _body,
        out_type=jax.ShapeDtypeStruct((SEQ, 8, NW, 8, CH), jnp.float32),
        mesh=mesh,
        compiler_params=pltpu.CompilerParams(
            needs_layout_passes=False, use_tc_tiling_on_sc=False
        ),
        scratch_types=(
            [pltpu.VMEM((NCH, CH), jnp.int32)]
            + [pltpu.VMEM((CH, HID), jnp.float32) for _ in range(NBUF)]
            + [pltpu.VMEM((8, 8, CH + 1), jnp.float32) for _ in range(NBUF)]
            + [pltpu.VMEM((2 * 256,), jnp.float32)]
            + [pltpu.SemaphoreType.DMA for _ in range(2 * NBUF)]
        ),
    )
    return f(idx, table)


def kernel(input_ids, weight):
    idx = input_ids.astype(jnp.int32).T.reshape(SEQ, NW, CH).transpose(1, 0, 2)
    wp = jnp.pad(weight, ((0, 0), (0, HIDP - HID))).reshape(-1, HID)
    idx = idx * 2
    out5 = _emb_call(idx, wp)
    return out5.transpose(2, 4, 0, 1, 3).reshape(B_TOK, SEQ, HID)
